# baseline (speedup 1.0000x reference)
import jax
import jax.numpy as jnp
from jax import lax
from jax.experimental import pallas as pl
from jax.experimental.pallas import tpu as pltpu
from jax.experimental.pallas import tpu_sc as plsc
import functools

N = 4096
L = 64
STEP_DIM = 2048
COMB_DIM = 1024
BK = 1024
NK = 2 * STEP_DIM // BK
NSTEP = NK // 2
JH = STEP_DIM // BK
VL = 16
NSL = N // VL
LSL = L // VL


def _sc_sample(adj_hbm, feats_hbm, labels_hbm, att_hbm, node1_hbm,
               node_hbm, u_hbm, scores_hbm,
               label_out, newnode_out, attsc_out,
               idx1, rowv, featv, labv, scov, attv, nodev, uv,
               scrf, scri, oi1, oi2, of1, sem):
    cid = lax.axis_index("c")
    sid = lax.axis_index("s")

    @pl.when((cid == 0) & (sid == 0))
    def _():
        iota = lax.iota(jnp.int32, VL)

        def rsum_f(x):
            for sh in (8, 4, 2, 1):
                scrf[...] = x
                x = x + plsc.load_gather(scrf, [jnp.bitwise_xor(iota, sh)])
            return x

        def rsum_i(x):
            for sh in (8, 4, 2, 1):
                scri[...] = x
                x = x + plsc.load_gather(scri, [jnp.bitwise_xor(iota, sh)])
            return x

        def rmax_f(x):
            for sh in (8, 4, 2, 1):
                scrf[...] = x
                x = jnp.maximum(x, plsc.load_gather(
                    scrf, [jnp.bitwise_xor(iota, sh)]))
            return x

        def rmin_f(x):
            for sh in (8, 4, 2, 1):
                scrf[...] = x
                x = jnp.minimum(x, plsc.load_gather(
                    scrf, [jnp.bitwise_xor(iota, sh)]))
            return x

        def prefix_f(x):
            for sh in (1, 2, 4, 8):
                scrf[...] = x
                y = plsc.load_gather(scrf, [jnp.maximum(iota - sh, 0)])
                x = x + jnp.where(iota >= sh, y, 0.0)
            return x

        def lane_last(x):
            scrf[...] = x
            return plsc.load_gather(scrf, [jnp.full((VL,), VL - 1, jnp.int32)])

        pltpu.sync_copy(node1_hbm, idx1)
        pltpu.sync_copy(node_hbm, nodev)
        pltpu.sync_copy(u_hbm, uv)
        pltpu.async_copy(adj_hbm.at[idx1], rowv, sem).wait()
        pltpu.sync_copy(feats_hbm, featv)
        pltpu.sync_copy(labels_hbm, labv)
        pltpu.sync_copy(scores_hbm, scov)
        pltpu.sync_copy(att_hbm, attv)
        uvec = uv[...]

        def nf_body(j, acc):
            sl = pl.ds(j * VL, VL)
            return acc + rowv[0, sl] * featv[sl]
        nfv = rsum_f(lax.fori_loop(0, NSL, nf_body,
                                   jnp.zeros((VL,), jnp.float32)))

        sacc = jnp.zeros((VL,), jnp.float32)
        for j in range(LSL):
            sacc = sacc + attv[pl.ds(j * VL, VL)] * nfv
        sv = rsum_f(sacc)
        carry = jnp.zeros((VL,), jnp.float32)
        cums = []
        for j in range(LSL):
            norm = attv[pl.ds(j * VL, VL)] * nfv / sv
            cum = prefix_f(norm) + carry
            cums.append(cum)
            carry = lane_last(cum)
        rv = carry * (1.0 - uvec)
        cnt = jnp.zeros((VL,), jnp.int32)
        for j in range(LSL):
            cnt = cnt + (cums[j] < rv).astype(jnp.int32)
        labelv = rsum_i(cnt)

        asc = jnp.zeros((VL,), jnp.float32)
        for j in range(LSL):
            hit = (iota + j * VL) == labelv
            asc = asc + jnp.where(hit, attv[pl.ds(j * VL, VL)], 0.0)
        ascv = rsum_f(asc)

        def cand_body(j, c):
            best, bidx, ccnt = c
            sl = pl.ds(j * VL, VL)
            cand = (rowv[0, sl] > 0.0) & (labv[sl] == labelv)
            sc = jnp.where(cand, scov[sl], -jnp.inf)
            upd = sc > best
            best = jnp.where(upd, sc, best)
            bidx = jnp.where(upd, (iota + j * VL).astype(jnp.float32), bidx)
            return best, bidx, ccnt + cand.astype(jnp.int32)
        best, bidx, ccnt = lax.fori_loop(
            0, NSL, cand_body,
            (jnp.full((VL,), -jnp.inf, jnp.float32),
             jnp.zeros((VL,), jnp.float32),
             jnp.zeros((VL,), jnp.int32)))
        mv = rmax_f(best)
        firstv = rmin_f(jnp.where(best == mv, bidx, jnp.float32(N)))
        ncandv = rsum_i(ccnt)
        newnodev = jnp.where(ncandv > 0, firstv.astype(jnp.int32),
                             nodev[...])

        oi1[...] = labelv
        oi2[...] = newnodev
        of1[...] = ascv
        pltpu.sync_copy(oi1, label_out)
        pltpu.sync_copy(oi2, newnode_out)
        pltpu.sync_copy(of1, attsc_out)


def _tc_state(lab_sref, att_ref, t1_ref, t2_ref, t3a_ref, t3b_ref,
              state_ref, comb_ref):
    k = pl.program_id(0)

    @pl.when(k == 0)
    def _build_comb():
        att_row = att_ref[0:1, :]
        for j in range(JH):
            comb_ref[j:j + 1, :] = jnp.dot(
                att_row, t1_ref[:, j * BK:(j + 1) * BK],
                preferred_element_type=jnp.float32)
        sub = lab_sref[0] % 8
        for j in range(JH):
            comb_ref[JH + j:JH + j + 1, :] = t2_ref[pl.ds(sub, 1),
                                                    j * BK:(j + 1) * BK]

    part = jnp.dot(comb_ref[pl.ds(k, 1), :], t3a_ref[...],
                   preferred_element_type=jnp.float32)
    part += jnp.dot(comb_ref[pl.ds(k + NSTEP, 1), :], t3b_ref[...],
                    preferred_element_type=jnp.float32)

    @pl.when(k == 0)
    def _init():
        state_ref[...] = part

    @pl.when(k > 0)
    def _acc():
        state_ref[...] += part


def kernel(adj, features, node_labels, node, attention,
           theta_step_1, theta_step_2, theta_step_3):
    key = jax.random.key(42)
    k1, k2 = jax.random.split(key)
    u1 = jax.random.uniform(k1, ())
    scores = jax.random.uniform(k2, (N,))

    node_i = jnp.asarray(node, jnp.int32)
    node1 = node_i.reshape((1,))
    node16 = jnp.full((VL,), node_i)
    u16 = jnp.full((VL,), u1, jnp.float32)
    labels_i = node_labels.astype(jnp.int32)

    mesh = plsc.VectorSubcoreMesh(core_axis_name="c", subcore_axis_name="s")
    sc = functools.partial(
        pl.kernel, mesh=mesh,
        compiler_params=pltpu.CompilerParams(use_tc_tiling_on_sc=False, needs_layout_passes=False),
        out_type=[
            jax.ShapeDtypeStruct((VL,), jnp.int32),
            jax.ShapeDtypeStruct((VL,), jnp.int32),
            jax.ShapeDtypeStruct((VL,), jnp.float32),
        ],
        scratch_types=[
            pltpu.VMEM((1,), jnp.int32),
            pltpu.VMEM((1, N), jnp.float32),
            pltpu.VMEM((N,), jnp.float32),
            pltpu.VMEM((N,), jnp.int32),
            pltpu.VMEM((N,), jnp.float32),
            pltpu.VMEM((L,), jnp.float32),
            pltpu.VMEM((VL,), jnp.int32),
            pltpu.VMEM((VL,), jnp.float32),
            pltpu.VMEM((VL,), jnp.float32),
            pltpu.VMEM((VL,), jnp.int32),
            pltpu.VMEM((VL,), jnp.int32),
            pltpu.VMEM((VL,), jnp.int32),
            pltpu.VMEM((VL,), jnp.float32),
            pltpu.SemaphoreType.DMA,
        ],
    )(_sc_sample)
    label16, newnode16, attsc16 = sc(
        adj, features, labels_i, attention, node1, node16, u16, scores)

    lab_arr = label16[0:1]

    grid_spec = pltpu.PrefetchScalarGridSpec(
        num_scalar_prefetch=1,
        grid=(NSTEP,),
        in_specs=[
            pl.BlockSpec((1, L), lambda k, n: (0, 0)),
            pl.BlockSpec((L, STEP_DIM), lambda k, n: (0, 0)),
            pl.BlockSpec((8, STEP_DIM), lambda k, n: (n[0] // 8, 0)),
            pl.BlockSpec((BK, COMB_DIM), lambda k, n: (k, 0)),
            pl.BlockSpec((BK, COMB_DIM), lambda k, n: (k + NSTEP, 0)),
        ],
        out_specs=pl.BlockSpec((1, COMB_DIM), lambda k, n: (0, 0)),
        scratch_shapes=[pltpu.VMEM((NK, BK), jnp.float32)],
    )
    state = pl.pallas_call(
        _tc_state,
        grid_spec=grid_spec,
        out_shape=jax.ShapeDtypeStruct((1, COMB_DIM), jnp.float32),
    )(lab_arr, attention.reshape(1, L), theta_step_1, theta_step_2,
      theta_step_3, theta_step_3)

    return (state.reshape(1, 1, COMB_DIM),
            newnode16[0].reshape(()),
            attsc16[0].reshape(()))


# speedup vs baseline: 1.0108x; 1.0108x over previous
import jax
import jax.numpy as jnp
from jax import lax
from jax.experimental import pallas as pl
from jax.experimental.pallas import tpu as pltpu
from jax.experimental.pallas import tpu_sc as plsc
import functools

N = 4096
L = 64
STEP_DIM = 2048
COMB_DIM = 1024
BK = 1024
NK = 2 * STEP_DIM // BK
NSTEP = NK // 2
JH = STEP_DIM // BK
VL = 16
NSL = N // VL
LSL = L // VL


def _sc_sample(adj_hbm, feats_hbm, labels_hbm, att_hbm, node1_hbm,
               node_hbm, u_hbm, scores_hbm,
               label_out, newnode_out, attsc_out,
               idx1, rowv, featv, labv, scov, attv, nodev, uv,
               scrf, scri, oi1, oi2, of1, sem):
    cid = lax.axis_index("c")
    sid = lax.axis_index("s")

    @pl.when((cid == 0) & (sid == 0))
    def _():
        iota = lax.iota(jnp.int32, VL)

        def rsum_f(x):
            for sh in (8, 4, 2, 1):
                scrf[...] = x
                x = x + plsc.load_gather(scrf, [jnp.bitwise_xor(iota, sh)])
            return x

        def rsum_i(x):
            for sh in (8, 4, 2, 1):
                scri[...] = x
                x = x + plsc.load_gather(scri, [jnp.bitwise_xor(iota, sh)])
            return x

        def rmax_f(x):
            for sh in (8, 4, 2, 1):
                scrf[...] = x
                x = jnp.maximum(x, plsc.load_gather(
                    scrf, [jnp.bitwise_xor(iota, sh)]))
            return x

        def rmin_f(x):
            for sh in (8, 4, 2, 1):
                scrf[...] = x
                x = jnp.minimum(x, plsc.load_gather(
                    scrf, [jnp.bitwise_xor(iota, sh)]))
            return x

        def prefix_f(x):
            for sh in (1, 2, 4, 8):
                scrf[...] = x
                y = plsc.load_gather(scrf, [jnp.maximum(iota - sh, 0)])
                x = x + jnp.where(iota >= sh, y, 0.0)
            return x

        def lane_last(x):
            scrf[...] = x
            return plsc.load_gather(scrf, [jnp.full((VL,), VL - 1, jnp.int32)])

        pltpu.sync_copy(node1_hbm, idx1)
        pltpu.sync_copy(node_hbm, nodev)
        pltpu.sync_copy(u_hbm, uv)
        pltpu.async_copy(adj_hbm.at[idx1], rowv, sem).wait()
        pltpu.sync_copy(feats_hbm, featv)
        pltpu.sync_copy(labels_hbm, labv)
        pltpu.sync_copy(scores_hbm, scov)
        pltpu.sync_copy(att_hbm, attv)
        uvec = uv[...]

        def nf_body(j, acc):
            sl = pl.ds(j * VL, VL)
            return acc + rowv[0, sl] * featv[sl]
        nfv = rsum_f(lax.fori_loop(0, NSL, nf_body,
                                   jnp.zeros((VL,), jnp.float32),
                                   unroll=8))

        sacc = jnp.zeros((VL,), jnp.float32)
        for j in range(LSL):
            sacc = sacc + attv[pl.ds(j * VL, VL)] * nfv
        sv = rsum_f(sacc)
        carry = jnp.zeros((VL,), jnp.float32)
        cums = []
        for j in range(LSL):
            norm = attv[pl.ds(j * VL, VL)] * nfv / sv
            cum = prefix_f(norm) + carry
            cums.append(cum)
            carry = lane_last(cum)
        rv = carry * (1.0 - uvec)
        cnt = jnp.zeros((VL,), jnp.int32)
        for j in range(LSL):
            cnt = cnt + (cums[j] < rv).astype(jnp.int32)
        labelv = rsum_i(cnt)

        asc = jnp.zeros((VL,), jnp.float32)
        for j in range(LSL):
            hit = (iota + j * VL) == labelv
            asc = asc + jnp.where(hit, attv[pl.ds(j * VL, VL)], 0.0)
        ascv = rsum_f(asc)

        def cand_body(j, c):
            best, bidx, ccnt = c
            sl = pl.ds(j * VL, VL)
            cand = (rowv[0, sl] > 0.0) & (labv[sl] == labelv)
            sc = jnp.where(cand, scov[sl], -jnp.inf)
            upd = sc > best
            best = jnp.where(upd, sc, best)
            bidx = jnp.where(upd, (iota + j * VL).astype(jnp.float32), bidx)
            return best, bidx, ccnt + cand.astype(jnp.int32)
        best, bidx, ccnt = lax.fori_loop(
            0, NSL, cand_body,
            (jnp.full((VL,), -jnp.inf, jnp.float32),
             jnp.zeros((VL,), jnp.float32),
             jnp.zeros((VL,), jnp.int32)), unroll=8)
        mv = rmax_f(best)
        firstv = rmin_f(jnp.where(best == mv, bidx, jnp.float32(N)))
        ncandv = rsum_i(ccnt)
        newnodev = jnp.where(ncandv > 0, firstv.astype(jnp.int32),
                             nodev[...])

        oi1[...] = labelv
        oi2[...] = newnodev
        of1[...] = ascv
        pltpu.sync_copy(oi1, label_out)
        pltpu.sync_copy(oi2, newnode_out)
        pltpu.sync_copy(of1, attsc_out)


def _tc_state(lab_sref, att_ref, t1_ref, t2_ref, t3a_ref, t3b_ref,
              state_ref, comb_ref):
    k = pl.program_id(0)

    @pl.when(k == 0)
    def _build_comb():
        att_row = att_ref[0:1, :]
        for j in range(JH):
            comb_ref[j:j + 1, :] = jnp.dot(
                att_row, t1_ref[:, j * BK:(j + 1) * BK],
                preferred_element_type=jnp.float32)
        sub = lab_sref[0] % 8
        for j in range(JH):
            comb_ref[JH + j:JH + j + 1, :] = t2_ref[pl.ds(sub, 1),
                                                    j * BK:(j + 1) * BK]

    part = jnp.dot(comb_ref[pl.ds(k, 1), :], t3a_ref[...],
                   preferred_element_type=jnp.float32)
    part += jnp.dot(comb_ref[pl.ds(k + NSTEP, 1), :], t3b_ref[...],
                    preferred_element_type=jnp.float32)

    @pl.when(k == 0)
    def _init():
        state_ref[...] = part

    @pl.when(k > 0)
    def _acc():
        state_ref[...] += part


def kernel(adj, features, node_labels, node, attention,
           theta_step_1, theta_step_2, theta_step_3):
    key = jax.random.key(42)
    k1, k2 = jax.random.split(key)
    u1 = jax.random.uniform(k1, ())
    scores = jax.random.uniform(k2, (N,))

    node_i = jnp.asarray(node, jnp.int32)
    node1 = node_i.reshape((1,))
    node16 = jnp.full((VL,), node_i)
    u16 = jnp.full((VL,), u1, jnp.float32)
    labels_i = node_labels.astype(jnp.int32)

    mesh = plsc.VectorSubcoreMesh(core_axis_name="c", subcore_axis_name="s")
    sc = functools.partial(
        pl.kernel, mesh=mesh,
        compiler_params=pltpu.CompilerParams(use_tc_tiling_on_sc=False, needs_layout_passes=False),
        out_type=[
            jax.ShapeDtypeStruct((VL,), jnp.int32),
            jax.ShapeDtypeStruct((VL,), jnp.int32),
            jax.ShapeDtypeStruct((VL,), jnp.float32),
        ],
        scratch_types=[
            pltpu.VMEM((1,), jnp.int32),
            pltpu.VMEM((1, N), jnp.float32),
            pltpu.VMEM((N,), jnp.float32),
            pltpu.VMEM((N,), jnp.int32),
            pltpu.VMEM((N,), jnp.float32),
            pltpu.VMEM((L,), jnp.float32),
            pltpu.VMEM((VL,), jnp.int32),
            pltpu.VMEM((VL,), jnp.float32),
            pltpu.VMEM((VL,), jnp.float32),
            pltpu.VMEM((VL,), jnp.int32),
            pltpu.VMEM((VL,), jnp.int32),
            pltpu.VMEM((VL,), jnp.int32),
            pltpu.VMEM((VL,), jnp.float32),
            pltpu.SemaphoreType.DMA,
        ],
    )(_sc_sample)
    label16, newnode16, attsc16 = sc(
        adj, features, labels_i, attention, node1, node16, u16, scores)

    lab_arr = label16[0:1]

    grid_spec = pltpu.PrefetchScalarGridSpec(
        num_scalar_prefetch=1,
        grid=(NSTEP,),
        in_specs=[
            pl.BlockSpec((1, L), lambda k, n: (0, 0)),
            pl.BlockSpec((L, STEP_DIM), lambda k, n: (0, 0)),
            pl.BlockSpec((8, STEP_DIM), lambda k, n: (n[0] // 8, 0)),
            pl.BlockSpec((BK, COMB_DIM), lambda k, n: (k, 0)),
            pl.BlockSpec((BK, COMB_DIM), lambda k, n: (k + NSTEP, 0)),
        ],
        out_specs=pl.BlockSpec((1, COMB_DIM), lambda k, n: (0, 0)),
        scratch_shapes=[pltpu.VMEM((NK, BK), jnp.float32)],
    )
    state = pl.pallas_call(
        _tc_state,
        grid_spec=grid_spec,
        out_shape=jax.ShapeDtypeStruct((1, COMB_DIM), jnp.float32),
    )(lab_arr, attention.reshape(1, L), theta_step_1, theta_step_2,
      theta_step_3, theta_step_3)

    return (state.reshape(1, 1, COMB_DIM),
            newnode16[0].reshape(()),
            attsc16[0].reshape(()))


# speedup vs baseline: 1.8720x; 1.8521x over previous
import jax
import jax.numpy as jnp
from jax import lax
from jax.experimental import pallas as pl
from jax.experimental.pallas import tpu as pltpu
from jax.experimental.pallas import tpu_sc as plsc
import functools

N = 4096
L = 64
STEP_DIM = 2048
COMB_DIM = 1024
BK = 1024
NK = 2 * STEP_DIM // BK
NSTEP = NK // 2
JH = STEP_DIM // BK
VL = 16
NSL = N // VL
LSL = L // VL


def _sc_sample(adj_hbm, feats_hbm, labels_hbm, att_hbm, node1_hbm,
               node_hbm, u_hbm, scores_hbm,
               label_out, newnode_out, attsc_out,
               idx1, rowv, featv, labv, scov, attv, nodev, uv,
               scrf, scri, oi1, oi2, of1, sem):
    cid = lax.axis_index("c")
    sid = lax.axis_index("s")

    @pl.when((cid == 0) & (sid == 0))
    def _():
        iota = lax.iota(jnp.int32, VL)

        def rsum_f(x):
            for sh in (8, 4, 2, 1):
                scrf[...] = x
                x = x + plsc.load_gather(scrf, [jnp.bitwise_xor(iota, sh)])
            return x

        def rsum_i(x):
            for sh in (8, 4, 2, 1):
                scri[...] = x
                x = x + plsc.load_gather(scri, [jnp.bitwise_xor(iota, sh)])
            return x

        def rmax_f(x):
            for sh in (8, 4, 2, 1):
                scrf[...] = x
                x = jnp.maximum(x, plsc.load_gather(
                    scrf, [jnp.bitwise_xor(iota, sh)]))
            return x

        def rmin_f(x):
            for sh in (8, 4, 2, 1):
                scrf[...] = x
                x = jnp.minimum(x, plsc.load_gather(
                    scrf, [jnp.bitwise_xor(iota, sh)]))
            return x

        def prefix_f(x):
            for sh in (1, 2, 4, 8):
                scrf[...] = x
                y = plsc.load_gather(scrf, [jnp.maximum(iota - sh, 0)])
                x = x + jnp.where(iota >= sh, y, 0.0)
            return x

        def lane_last(x):
            scrf[...] = x
            return plsc.load_gather(scrf, [jnp.full((VL,), VL - 1, jnp.int32)])

        pltpu.sync_copy(node1_hbm, idx1)
        pltpu.sync_copy(node_hbm, nodev)
        pltpu.sync_copy(u_hbm, uv)
        node_s = nodev[...][0]
        pltpu.sync_copy(adj_hbm.at[pl.ds(node_s, 1), :], rowv)
        pltpu.sync_copy(feats_hbm, featv)
        pltpu.sync_copy(labels_hbm, labv)
        pltpu.sync_copy(scores_hbm, scov)
        pltpu.sync_copy(att_hbm, attv)
        uvec = uv[...]

        def nf_body(j, acc):
            sl = pl.ds(j * VL, VL)
            return acc + rowv[0, sl] * featv[sl]
        nfv = rsum_f(lax.fori_loop(0, NSL, nf_body,
                                   jnp.zeros((VL,), jnp.float32),
                                   unroll=8))

        sacc = jnp.zeros((VL,), jnp.float32)
        for j in range(LSL):
            sacc = sacc + attv[pl.ds(j * VL, VL)] * nfv
        sv = rsum_f(sacc)
        carry = jnp.zeros((VL,), jnp.float32)
        cums = []
        for j in range(LSL):
            norm = attv[pl.ds(j * VL, VL)] * nfv / sv
            cum = prefix_f(norm) + carry
            cums.append(cum)
            carry = lane_last(cum)
        rv = carry * (1.0 - uvec)
        cnt = jnp.zeros((VL,), jnp.int32)
        for j in range(LSL):
            cnt = cnt + (cums[j] < rv).astype(jnp.int32)
        labelv = rsum_i(cnt)

        asc = jnp.zeros((VL,), jnp.float32)
        for j in range(LSL):
            hit = (iota + j * VL) == labelv
            asc = asc + jnp.where(hit, attv[pl.ds(j * VL, VL)], 0.0)
        ascv = rsum_f(asc)

        def cand_body(j, c):
            best, bidx, ccnt = c
            sl = pl.ds(j * VL, VL)
            cand = (rowv[0, sl] > 0.0) & (labv[sl] == labelv)
            sc = jnp.where(cand, scov[sl], -jnp.inf)
            upd = sc > best
            best = jnp.where(upd, sc, best)
            bidx = jnp.where(upd, (iota + j * VL).astype(jnp.float32), bidx)
            return best, bidx, ccnt + cand.astype(jnp.int32)
        best, bidx, ccnt = lax.fori_loop(
            0, NSL, cand_body,
            (jnp.full((VL,), -jnp.inf, jnp.float32),
             jnp.zeros((VL,), jnp.float32),
             jnp.zeros((VL,), jnp.int32)), unroll=8)
        mv = rmax_f(best)
        firstv = rmin_f(jnp.where(best == mv, bidx, jnp.float32(N)))
        ncandv = rsum_i(ccnt)
        newnodev = jnp.where(ncandv > 0, firstv.astype(jnp.int32),
                             nodev[...])

        oi1[...] = labelv
        oi2[...] = newnodev
        of1[...] = ascv
        pltpu.sync_copy(oi1, label_out)
        pltpu.sync_copy(oi2, newnode_out)
        pltpu.sync_copy(of1, attsc_out)


def _tc_state(lab_sref, att_ref, t1_ref, t2_ref, t3a_ref, t3b_ref,
              state_ref, comb_ref):
    k = pl.program_id(0)

    @pl.when(k == 0)
    def _build_comb():
        att_row = att_ref[0:1, :]
        for j in range(JH):
            comb_ref[j:j + 1, :] = jnp.dot(
                att_row, t1_ref[:, j * BK:(j + 1) * BK],
                preferred_element_type=jnp.float32)
        sub = lab_sref[0] % 8
        for j in range(JH):
            comb_ref[JH + j:JH + j + 1, :] = t2_ref[pl.ds(sub, 1),
                                                    j * BK:(j + 1) * BK]

    part = jnp.dot(comb_ref[pl.ds(k, 1), :], t3a_ref[...],
                   preferred_element_type=jnp.float32)
    part += jnp.dot(comb_ref[pl.ds(k + NSTEP, 1), :], t3b_ref[...],
                    preferred_element_type=jnp.float32)

    @pl.when(k == 0)
    def _init():
        state_ref[...] = part

    @pl.when(k > 0)
    def _acc():
        state_ref[...] += part


def kernel(adj, features, node_labels, node, attention,
           theta_step_1, theta_step_2, theta_step_3):
    key = jax.random.key(42)
    k1, k2 = jax.random.split(key)
    u1 = jax.random.uniform(k1, ())
    scores = jax.random.uniform(k2, (N,))

    node_i = jnp.asarray(node, jnp.int32)
    node1 = node_i.reshape((1,))
    node16 = jnp.full((VL,), node_i)
    u16 = jnp.full((VL,), u1, jnp.float32)
    labels_i = node_labels.astype(jnp.int32)

    mesh = plsc.VectorSubcoreMesh(core_axis_name="c", subcore_axis_name="s")
    sc = functools.partial(
        pl.kernel, mesh=mesh,
        compiler_params=pltpu.CompilerParams(use_tc_tiling_on_sc=True, needs_layout_passes=False),
        out_type=[
            jax.ShapeDtypeStruct((VL,), jnp.int32),
            jax.ShapeDtypeStruct((VL,), jnp.int32),
            jax.ShapeDtypeStruct((VL,), jnp.float32),
        ],
        scratch_types=[
            pltpu.VMEM((1,), jnp.int32),
            pltpu.VMEM((1, N), jnp.float32),
            pltpu.VMEM((N,), jnp.float32),
            pltpu.VMEM((N,), jnp.int32),
            pltpu.VMEM((N,), jnp.float32),
            pltpu.VMEM((L,), jnp.float32),
            pltpu.VMEM((VL,), jnp.int32),
            pltpu.VMEM((VL,), jnp.float32),
            pltpu.VMEM((VL,), jnp.float32),
            pltpu.VMEM((VL,), jnp.int32),
            pltpu.VMEM((VL,), jnp.int32),
            pltpu.VMEM((VL,), jnp.int32),
            pltpu.VMEM((VL,), jnp.float32),
            pltpu.SemaphoreType.DMA,
        ],
    )(_sc_sample)
    label16, newnode16, attsc16 = sc(
        adj, features, labels_i, attention, node1, node16, u16, scores)

    lab_arr = label16[0:1]

    grid_spec = pltpu.PrefetchScalarGridSpec(
        num_scalar_prefetch=1,
        grid=(NSTEP,),
        in_specs=[
            pl.BlockSpec((1, L), lambda k, n: (0, 0)),
            pl.BlockSpec((L, STEP_DIM), lambda k, n: (0, 0)),
            pl.BlockSpec((8, STEP_DIM), lambda k, n: (n[0] // 8, 0)),
            pl.BlockSpec((BK, COMB_DIM), lambda k, n: (k, 0)),
            pl.BlockSpec((BK, COMB_DIM), lambda k, n: (k + NSTEP, 0)),
        ],
        out_specs=pl.BlockSpec((1, COMB_DIM), lambda k, n: (0, 0)),
        scratch_shapes=[pltpu.VMEM((NK, BK), jnp.float32)],
    )
    state = pl.pallas_call(
        _tc_state,
        grid_spec=grid_spec,
        out_shape=jax.ShapeDtypeStruct((1, COMB_DIM), jnp.float32),
    )(lab_arr, attention.reshape(1, L), theta_step_1, theta_step_2,
      theta_step_3, theta_step_3)

    return (state.reshape(1, 1, COMB_DIM),
            newnode16[0].reshape(()),
            attsc16[0].reshape(()))
